# baseline (speedup 1.0000x reference)
import jax
import jax.numpy as jnp
from jax.experimental import pallas as pl


def _body(pre_ref, ctx_ref, suf_ref, out_ref):
    b, _, d = pre_ref.shape
    n_ctx = ctx_ref.shape[0]
    s = suf_ref.shape[1]
    out_ref[:, 0:1, :] = pre_ref[...]
    out_ref[:, 1:1 + n_ctx, :] = jnp.broadcast_to(ctx_ref[...][None], (b, n_ctx, d))
    out_ref[:, 1 + n_ctx:1 + n_ctx + s, :] = suf_ref[...]
    out_ref[:, 1 + n_ctx + s:, :] = jnp.zeros_like(out_ref[:, 1 + n_ctx + s:, :])


def kernel(ctx, token_prefix, token_suffix):
    n_cls, _, d = token_prefix.shape
    n_ctx = ctx.shape[0]
    s = token_suffix.shape[1]
    seq = 1 + n_ctx + s
    seq_pad = 80

    B = 50
    out = pl.pallas_call(
        _body,
        grid=(n_cls // B,),
        in_specs=[
            pl.BlockSpec((B, 1, d), lambda i: (i, 0, 0)),
            pl.BlockSpec((n_ctx, d), lambda i: (0, 0)),
            pl.BlockSpec((B, s, d), lambda i: (i, 0, 0)),
        ],
        out_specs=pl.BlockSpec((B, seq_pad, d), lambda i: (i, 0, 0)),
        out_shape=jax.ShapeDtypeStruct((n_cls, seq_pad, d), jnp.float32),
    )(token_prefix, ctx, token_suffix)
    return out[:, :seq, :]


# speedup vs baseline: 3.1528x; 3.1528x over previous
import jax
import jax.numpy as jnp
from jax.experimental import pallas as pl


def _body(pre_ref, ctx_ref, suf_ref, out_ref):
    b, _, d = pre_ref.shape
    n_ctx = ctx_ref.shape[0]
    s = suf_ref.shape[1]
    out_ref[:, 0:1, :] = pre_ref[...]
    out_ref[:, 1:1 + n_ctx, :] = jnp.broadcast_to(ctx_ref[...][None], (b, n_ctx, d))
    out_ref[:, 1 + n_ctx:, :] = suf_ref[...]


def kernel(ctx, token_prefix, token_suffix):
    n_cls, _, d = token_prefix.shape
    n_ctx = ctx.shape[0]
    s = token_suffix.shape[1]
    seq = 1 + n_ctx + s

    B = 50
    return pl.pallas_call(
        _body,
        grid=(n_cls // B,),
        in_specs=[
            pl.BlockSpec((B, 1, d), lambda i: (i, 0, 0)),
            pl.BlockSpec((n_ctx, d), lambda i: (0, 0)),
            pl.BlockSpec((B, s, d), lambda i: (i, 0, 0)),
        ],
        out_specs=pl.BlockSpec((B, seq, d), lambda i: (i, 0, 0)),
        out_shape=jax.ShapeDtypeStruct((n_cls, seq, d), jnp.float32),
    )(token_prefix, ctx, token_suffix)
